# baseline (speedup 1.0000x reference)
import functools

import jax
import jax.numpy as jnp
from jax import lax
from jax.experimental import pallas as pl
from jax.experimental.pallas import tpu as pltpu
from jax.experimental.pallas import tpu_sc as plsc

N = 10000
IN_DIM = 128
D = 64
E = 320000
NG = 64

NC = 2
NS = 16
NW = NC * NS
K = 128
CH = 79
EPAD = NW * CH * K
NP = 10240
RPT = NP // NS

BN = 2048
BP = BN // 2
GN = NP // BN
NPP = NP // 2

_mesh = plsc.VectorSubcoreMesh(core_axis_name="c", subcore_axis_name="s")


@functools.partial(
    pl.kernel,
    mesh=_mesh,
    out_type=jax.ShapeDtypeStruct((NC, NP, D), jnp.float32),
    scratch_types=[
        pltpu.VMEM((CH, K), jnp.int32),
        pltpu.VMEM((K,), jnp.float32),
        pltpu.VMEM((RPT,), jnp.float32),
        pltpu.VMEM((RPT, D), jnp.float32),
        pltpu.VMEM_SHARED((NP,), jnp.float32),
    ],
    compiler_params=pltpu.CompilerParams(use_tc_tiling_on_sc=False),
)
def _sc_degree(e_hbm, ones_hbm, zeros_hbm, out_hbm,
               dst_v, ones_v, deg_v, wide_v, acc_sh):
    cid = lax.axis_index("c")
    sid = lax.axis_index("s")
    wid = cid * NS + sid
    pltpu.sync_copy(e_hbm.at[1, wid], dst_v)
    pltpu.sync_copy(ones_hbm, ones_v)
    pltpu.sync_copy(zeros_hbm.at[pl.ds(sid * RPT, RPT)],
                    acc_sh.at[pl.ds(sid * RPT, RPT)])
    plsc.subcore_barrier()

    def body(j, carry):
        pltpu.sync_copy(ones_v, acc_sh.at[dst_v.at[j]], add=True)
        return carry

    lax.fori_loop(0, CH, body, 0)
    plsc.subcore_barrier()

    pltpu.sync_copy(acc_sh.at[pl.ds(sid * RPT, RPT)], deg_v)

    def bodyb(t, carry):
        base = t * 16
        v = deg_v[pl.ds(base, 16)]
        for l in range(16):
            row = jnp.full((16,), v[l], jnp.float32)
            for c in range(4):
                wide_v[base + l, pl.ds(16 * c, 16)] = row
        return carry

    lax.fori_loop(0, RPT // 16, bodyb, 0)
    pltpu.sync_copy(wide_v, out_hbm.at[cid, pl.ds(sid * RPT, RPT)])


@functools.partial(
    pl.kernel,
    mesh=_mesh,
    out_type=jax.ShapeDtypeStruct((NC, NP, D), jnp.float32),
    scratch_types=[
        pltpu.VMEM((CH, K), jnp.int32),
        pltpu.VMEM((CH, K), jnp.int32),
        pltpu.VMEM((2, K, D), jnp.float32),
        pltpu.VMEM_SHARED((NP, D), jnp.float32),
        pltpu.SemaphoreType.DMA,
        pltpu.SemaphoreType.DMA,
    ],
    compiler_params=pltpu.CompilerParams(use_tc_tiling_on_sc=False),
)
def _sc_scatter(g_hbm, e_hbm, zeros_hbm, out_hbm,
                src_v, dst_v, rows_v, acc_sh, sem0, sem1):
    cid = lax.axis_index("c")
    sid = lax.axis_index("s")
    wid = cid * NS + sid
    pltpu.sync_copy(e_hbm.at[0, wid], src_v)
    pltpu.sync_copy(e_hbm.at[1, wid], dst_v)
    pltpu.sync_copy(zeros_hbm.at[pl.ds(sid * RPT, RPT)],
                    acc_sh.at[pl.ds(sid * RPT, RPT)])
    plsc.subcore_barrier()

    pltpu.async_copy(g_hbm.at[src_v.at[0]], rows_v.at[0], sem0)

    def body(t, carry):
        j = 2 * t
        pltpu.async_copy(g_hbm.at[src_v.at[j + 1]], rows_v.at[1], sem1)
        pltpu.make_async_copy(g_hbm.at[src_v.at[j]], rows_v.at[0], sem0).wait()
        pltpu.sync_copy(rows_v.at[0], acc_sh.at[dst_v.at[j]], add=True)

        @pl.when(j + 2 < CH)
        def _():
            pltpu.async_copy(g_hbm.at[src_v.at[j + 2]], rows_v.at[0], sem0)

        pltpu.make_async_copy(g_hbm.at[src_v.at[j + 1]], rows_v.at[1],
                              sem1).wait()
        pltpu.sync_copy(rows_v.at[1], acc_sh.at[dst_v.at[j + 1]], add=True)
        return carry

    lax.fori_loop(0, CH // 2, body, 0)
    pltpu.make_async_copy(g_hbm.at[src_v.at[CH - 1]], rows_v.at[0],
                          sem0).wait()
    pltpu.sync_copy(rows_v.at[0], acc_sh.at[dst_v.at[CH - 1]], add=True)

    plsc.subcore_barrier()
    pltpu.sync_copy(acc_sh.at[pl.ds(sid * RPT, RPT)],
                    out_hbm.at[cid, pl.ds(sid * RPT, RPT)])


def _dinv_packed(deg_ref):
    d = deg_ref[...]
    dd = d[0] + d[1] + 1.0
    return lax.rsqrt(jnp.maximum(dd, 1.0))


def _tc1_body(x_ref, w1_ref, deg_ref, g_ref):
    dinv = _dinv_packed(deg_ref)
    h = jnp.dot(x_ref[...], w1_ref[...], preferred_element_type=jnp.float32)
    g_ref[...] = h * dinv


_tc_stage1 = pl.pallas_call(
    _tc1_body,
    grid=(GN,),
    in_specs=[
        pl.BlockSpec((BP, 2 * IN_DIM), lambda i: (i, 0)),
        pl.BlockSpec((2 * IN_DIM, 128), lambda i: (0, 0)),
        pl.BlockSpec((NC, BP, 128), lambda i: (0, i, 0)),
    ],
    out_specs=pl.BlockSpec((BP, 128), lambda i: (i, 0)),
    out_shape=jax.ShapeDtypeStruct((NPP, 128), jnp.float32),
)


def _tc2_body(s_ref, g1_ref, deg_ref, w2_ref, b1_ref, g2_ref):
    dinv = _dinv_packed(deg_ref)
    sp = s_ref[...]
    s = sp[0] + sp[1]
    t1 = jnp.maximum((s + g1_ref[...]) * dinv + b1_ref[...], 0.0)
    h2 = jnp.dot(t1, w2_ref[...], preferred_element_type=jnp.float32)
    g2_ref[...] = h2 * dinv


_tc_stage2 = pl.pallas_call(
    _tc2_body,
    grid=(GN,),
    in_specs=[
        pl.BlockSpec((NC, BP, 128), lambda i: (0, i, 0)),
        pl.BlockSpec((BP, 128), lambda i: (i, 0)),
        pl.BlockSpec((NC, BP, 128), lambda i: (0, i, 0)),
        pl.BlockSpec((128, 128), lambda i: (0, 0)),
        pl.BlockSpec((1, 128), lambda i: (0, 0)),
    ],
    out_specs=pl.BlockSpec((BP, 128), lambda i: (i, 0)),
    out_shape=jax.ShapeDtypeStruct((NPP, 128), jnp.float32),
)


def _tc3_body(s_ref, g2_ref, deg_ref, b2_ref, wl_ref, bl_ref, be_ref, bo_ref,
              out_ref, acc_ref):
    i = pl.program_id(0)
    dinv = _dinv_packed(deg_ref)
    sp = s_ref[...]
    s = sp[0] + sp[1]
    h = jnp.maximum((s + g2_ref[...]) * dinv + b2_ref[...], 0.0)
    zp = jnp.dot(h, wl_ref[...], preferred_element_type=jnp.float32)
    ones = jnp.ones((BP, 1), jnp.float32)
    giota = lax.broadcasted_iota(jnp.int32, (BP, NG), 1)
    part = jnp.zeros((NG, 2), jnp.float32)
    for b_ref, col in ((be_ref, 0), (bo_ref, 1)):
        b = b_ref[0, 0, :]
        onehot = (b[:, None] == giota).astype(jnp.float32)
        vals = jnp.where(b[:, None] >= 0,
                         jnp.concatenate([zp[:, col:col + 1], ones], axis=1),
                         0.0)
        part = part + lax.dot_general(onehot, vals, (((0,), (0,)), ((), ())),
                                      preferred_element_type=jnp.float32)

    @pl.when(i == 0)
    def _():
        acc_ref[...] = jnp.zeros_like(acc_ref)

    acc_ref[...] += part
    out_ref[0, :] = (acc_ref[:, 0] / jnp.maximum(acc_ref[:, 1], 1.0)
                     + bl_ref[0, 0])


_tc_stage3 = pl.pallas_call(
    _tc3_body,
    grid=(GN,),
    in_specs=[
        pl.BlockSpec((NC, BP, 128), lambda i: (0, i, 0)),
        pl.BlockSpec((BP, 128), lambda i: (i, 0)),
        pl.BlockSpec((NC, BP, 128), lambda i: (0, i, 0)),
        pl.BlockSpec((1, 128), lambda i: (0, 0)),
        pl.BlockSpec((128, 2), lambda i: (0, 0)),
        pl.BlockSpec((1, 1), lambda i: (0, 0)),
        pl.BlockSpec((1, 1, BP), lambda i: (i, 0, 0)),
        pl.BlockSpec((1, 1, BP), lambda i: (i, 0, 0)),
    ],
    out_specs=pl.BlockSpec((1, NG), lambda i: (0, 0)),
    out_shape=jax.ShapeDtypeStruct((1, NG), jnp.float32),
    scratch_shapes=[pltpu.VMEM((NG, 2), jnp.float32)],
)


def _blockdiag2(w):
    r, c = w.shape
    z = jnp.zeros((r, c), w.dtype)
    return jnp.concatenate(
        [jnp.concatenate([w, z], axis=1), jnp.concatenate([z, w], axis=1)],
        axis=0)


def kernel(x, edge_index, batch, W1, b1, W2, b2, Wlin, blin):
    epad = jnp.full((2, EPAD - E), NP - 1, jnp.int32)
    e4 = jnp.concatenate([edge_index, epad], axis=1).reshape(2, NW, CH, K)
    ones1 = jnp.ones((K,), jnp.float32)
    zer1 = jnp.zeros((NP,), jnp.float32)
    zer64 = jnp.zeros((NP, D), jnp.float32)
    batch_pad = jnp.concatenate([batch, jnp.full((NP - N,), -1, jnp.int32)])
    be = batch_pad[0::2].reshape(GN, 1, BP)
    bo = batch_pad[1::2].reshape(GN, 1, BP)
    w1bd = _blockdiag2(W1)
    w2bd = _blockdiag2(W2)
    wlbd = _blockdiag2(Wlin)
    b1p = jnp.tile(b1, 2).reshape(1, 128)
    b2p = jnp.tile(b2, 2).reshape(1, 128)

    ddb = _sc_degree(e4, ones1, zer1)
    ddb_p = ddb.reshape(NC, NPP, 128)
    g1p = _tc_stage1(x.reshape(N // 2, 2 * IN_DIM), w1bd, ddb_p)
    s1 = _sc_scatter(g1p.reshape(NP, D), e4, zer64)
    g2p = _tc_stage2(s1.reshape(NC, NPP, 128), g1p, ddb_p, w2bd, b1p)
    s2 = _sc_scatter(g2p.reshape(NP, D), e4, zer64)
    out = _tc_stage3(s2.reshape(NC, NPP, 128), g2p, ddb_p, b2p, wlbd,
                     blin.reshape(1, 1), be, bo)
    return out.reshape(-1)


# speedup vs baseline: 1.5893x; 1.5893x over previous
import functools

import jax
import jax.numpy as jnp
from jax import lax
from jax.experimental import pallas as pl
from jax.experimental.pallas import tpu as pltpu
from jax.experimental.pallas import tpu_sc as plsc

N = 10000
IN_DIM = 128
D = 64
E = 320000
NG = 64

NC = 2
NS = 16
NW = NC * NS
EPT = E // NW
K = 80
CH = EPT // K
NP = 10240
RPT = NP // NS

BN = 2048
BP = BN // 2
GN = NP // BN
NPP = NP // 2
NPK = N * D // 128

_mesh = plsc.VectorSubcoreMesh(core_axis_name="c", subcore_axis_name="s")


@functools.partial(
    pl.kernel,
    mesh=_mesh,
    out_type=jax.ShapeDtypeStruct((NC, NP, D), jnp.float32),
    scratch_types=[
        pltpu.VMEM((CH, K), jnp.int32),
        pltpu.VMEM((K,), jnp.float32),
        pltpu.VMEM((RPT,), jnp.float32),
        pltpu.VMEM((RPT, D), jnp.float32),
        pltpu.VMEM_SHARED((NP,), jnp.float32),
    ],
    compiler_params=pltpu.CompilerParams(use_tc_tiling_on_sc=False),
)
def _sc_degree(e_hbm, ones_hbm, zeros_hbm, out_hbm,
               dst_v, ones_v, deg_v, wide_v, acc_sh):
    cid = lax.axis_index("c")
    sid = lax.axis_index("s")
    wid = cid * NS + sid
    pltpu.sync_copy(e_hbm.at[1, wid], dst_v)
    pltpu.sync_copy(ones_hbm, ones_v)
    pltpu.sync_copy(zeros_hbm.at[pl.ds(sid * RPT, RPT)],
                    acc_sh.at[pl.ds(sid * RPT, RPT)])
    plsc.subcore_barrier()

    def body(j, carry):
        pltpu.sync_copy(ones_v, acc_sh.at[dst_v.at[j]], add=True)
        return carry

    lax.fori_loop(0, CH, body, 0)
    plsc.subcore_barrier()

    pltpu.sync_copy(acc_sh.at[pl.ds(sid * RPT, RPT)], deg_v)

    def bodyb(t, carry):
        base = t * 16
        v = deg_v[pl.ds(base, 16)]
        for l in range(16):
            row = jnp.full((16,), v[l], jnp.float32)
            for c in range(4):
                wide_v[base + l, pl.ds(16 * c, 16)] = row
        return carry

    lax.fori_loop(0, RPT // 16, bodyb, 0)
    pltpu.sync_copy(wide_v, out_hbm.at[cid, pl.ds(sid * RPT, RPT)])


@functools.partial(
    pl.kernel,
    mesh=_mesh,
    out_type=jax.ShapeDtypeStruct((NC, NP, D), jnp.float32),
    scratch_types=[
        pltpu.VMEM((CH, K), jnp.int32),
        pltpu.VMEM((CH, K), jnp.int32),
        pltpu.VMEM((2, K, D), jnp.float32),
        pltpu.VMEM_SHARED((NP, D), jnp.float32),
        pltpu.SemaphoreType.DMA,
        pltpu.SemaphoreType.DMA,
    ],
    compiler_params=pltpu.CompilerParams(use_tc_tiling_on_sc=False),
)
def _sc_scatter(g_hbm, e_hbm, zeros_hbm, out_hbm,
                src_v, dst_v, rows_v, acc_sh, sem0, sem1):
    cid = lax.axis_index("c")
    sid = lax.axis_index("s")
    wid = cid * NS + sid
    pltpu.sync_copy(e_hbm.at[0, wid], src_v)
    pltpu.sync_copy(e_hbm.at[1, wid], dst_v)
    pltpu.sync_copy(zeros_hbm.at[pl.ds(sid * RPT, RPT)],
                    acc_sh.at[pl.ds(sid * RPT, RPT)])
    plsc.subcore_barrier()

    pltpu.async_copy(g_hbm.at[src_v.at[0]], rows_v.at[0], sem0)

    def body(t, carry):
        j = 2 * t
        pltpu.async_copy(g_hbm.at[src_v.at[j + 1]], rows_v.at[1], sem1)
        pltpu.make_async_copy(g_hbm.at[src_v.at[j]], rows_v.at[0], sem0).wait()
        pltpu.sync_copy(rows_v.at[0], acc_sh.at[dst_v.at[j]], add=True)

        @pl.when(j + 2 < CH)
        def _():
            pltpu.async_copy(g_hbm.at[src_v.at[j + 2]], rows_v.at[0], sem0)

        pltpu.make_async_copy(g_hbm.at[src_v.at[j + 1]], rows_v.at[1],
                              sem1).wait()
        pltpu.sync_copy(rows_v.at[1], acc_sh.at[dst_v.at[j + 1]], add=True)
        return carry

    lax.fori_loop(0, CH // 2, body, 0)
    pltpu.make_async_copy(g_hbm.at[src_v.at[CH - 1]], rows_v.at[0],
                          sem0).wait()
    pltpu.sync_copy(rows_v.at[0], acc_sh.at[dst_v.at[CH - 1]], add=True)

    plsc.subcore_barrier()
    pltpu.sync_copy(acc_sh.at[pl.ds(sid * RPT, RPT)],
                    out_hbm.at[cid, pl.ds(sid * RPT, RPT)])


def _dinv_packed(deg_ref):
    d = deg_ref[...]
    dd = d[0] + d[1] + 1.0
    return lax.rsqrt(jnp.maximum(dd, 1.0))


def _tc1_body(x_ref, w1_ref, deg_ref, g_ref):
    dinv = _dinv_packed(deg_ref)
    h = jnp.dot(x_ref[...], w1_ref[...], preferred_element_type=jnp.float32)
    g_ref[...] = h * dinv


_tc_stage1 = pl.pallas_call(
    _tc1_body,
    grid=(GN,),
    in_specs=[
        pl.BlockSpec((BP, 2 * IN_DIM), lambda i: (i, 0)),
        pl.BlockSpec((2 * IN_DIM, 128), lambda i: (0, 0)),
        pl.BlockSpec((NC, BP, 128), lambda i: (0, i, 0)),
    ],
    out_specs=pl.BlockSpec((BP, 128), lambda i: (i, 0)),
    out_shape=jax.ShapeDtypeStruct((NPK, 128), jnp.float32),
)


def _tc2_body(s_ref, g1_ref, deg_ref, w2_ref, b1_ref, g2_ref):
    dinv = _dinv_packed(deg_ref)
    sp = s_ref[...]
    s = sp[0] + sp[1]
    t1 = jnp.maximum((s + g1_ref[...]) * dinv + b1_ref[...], 0.0)
    h2 = jnp.dot(t1, w2_ref[...], preferred_element_type=jnp.float32)
    g2_ref[...] = h2 * dinv


_tc_stage2 = pl.pallas_call(
    _tc2_body,
    grid=(GN,),
    in_specs=[
        pl.BlockSpec((NC, BP, 128), lambda i: (0, i, 0)),
        pl.BlockSpec((BP, 128), lambda i: (i, 0)),
        pl.BlockSpec((NC, BP, 128), lambda i: (0, i, 0)),
        pl.BlockSpec((128, 128), lambda i: (0, 0)),
        pl.BlockSpec((1, 128), lambda i: (0, 0)),
    ],
    out_specs=pl.BlockSpec((BP, 128), lambda i: (i, 0)),
    out_shape=jax.ShapeDtypeStruct((NPK, 128), jnp.float32),
)


def _tc3_body(s_ref, g2_ref, deg_ref, b2_ref, wl_ref, bl_ref, be_ref, bo_ref,
              out_ref, acc_ref):
    i = pl.program_id(0)
    dinv = _dinv_packed(deg_ref)
    sp = s_ref[...]
    s = sp[0] + sp[1]
    h = jnp.maximum((s + g2_ref[...]) * dinv + b2_ref[...], 0.0)
    zp = jnp.dot(h, wl_ref[...], preferred_element_type=jnp.float32)
    ones = jnp.ones((BP, 1), jnp.float32)
    giota = lax.broadcasted_iota(jnp.int32, (BP, NG), 1)
    part = jnp.zeros((NG, 2), jnp.float32)
    for b_ref, col in ((be_ref, 0), (bo_ref, 1)):
        b = b_ref[0, 0, :]
        onehot = (b[:, None] == giota).astype(jnp.float32)
        vals = jnp.where(b[:, None] >= 0,
                         jnp.concatenate([zp[:, col:col + 1], ones], axis=1),
                         0.0)
        part = part + lax.dot_general(onehot, vals, (((0,), (0,)), ((), ())),
                                      preferred_element_type=jnp.float32)

    @pl.when(i == 0)
    def _():
        acc_ref[...] = jnp.zeros_like(acc_ref)

    acc_ref[...] += part
    out_ref[0, :] = (acc_ref[:, 0] / jnp.maximum(acc_ref[:, 1], 1.0)
                     + bl_ref[0, 0])


_tc_stage3 = pl.pallas_call(
    _tc3_body,
    grid=(GN,),
    in_specs=[
        pl.BlockSpec((NC, BP, 128), lambda i: (0, i, 0)),
        pl.BlockSpec((BP, 128), lambda i: (i, 0)),
        pl.BlockSpec((NC, BP, 128), lambda i: (0, i, 0)),
        pl.BlockSpec((1, 128), lambda i: (0, 0)),
        pl.BlockSpec((128, 2), lambda i: (0, 0)),
        pl.BlockSpec((1, 1), lambda i: (0, 0)),
        pl.BlockSpec((1, 1, BP), lambda i: (i, 0, 0)),
        pl.BlockSpec((1, 1, BP), lambda i: (i, 0, 0)),
    ],
    out_specs=pl.BlockSpec((1, NG), lambda i: (0, 0)),
    out_shape=jax.ShapeDtypeStruct((1, NG), jnp.float32),
    scratch_shapes=[pltpu.VMEM((NG, 2), jnp.float32)],
)


def _blockdiag2(w):
    r, c = w.shape
    z = jnp.zeros((r, c), w.dtype)
    return jnp.concatenate(
        [jnp.concatenate([w, z], axis=1), jnp.concatenate([z, w], axis=1)],
        axis=0)


def kernel(x, edge_index, batch, W1, b1, W2, b2, Wlin, blin):
    e4 = edge_index.reshape(2, NW, CH, K)
    ones1 = jnp.ones((K,), jnp.float32)
    zer1 = jnp.zeros((NP,), jnp.float32)
    zer64 = jnp.zeros((NP, D), jnp.float32)
    batch_pad = jnp.concatenate([batch, jnp.full((NP - N,), -1, jnp.int32)])
    be = batch_pad[0::2].reshape(GN, 1, BP)
    bo = batch_pad[1::2].reshape(GN, 1, BP)
    w1bd = _blockdiag2(W1)
    w2bd = _blockdiag2(W2)
    wlbd = _blockdiag2(Wlin)
    b1p = jnp.tile(b1, 2).reshape(1, 128)
    b2p = jnp.tile(b2, 2).reshape(1, 128)

    ddb = _sc_degree(e4, ones1, zer1)
    ddb_p = ddb.reshape(NC, NPP, 128)
    g1p = _tc_stage1(x.reshape(N // 2, 2 * IN_DIM), w1bd, ddb_p)
    s1 = _sc_scatter(g1p.reshape(N, D), e4, zer64)
    g2p = _tc_stage2(s1.reshape(NC, NPP, 128), g1p, ddb_p, w2bd, b1p)
    s2 = _sc_scatter(g2p.reshape(N, D), e4, zer64)
    out = _tc_stage3(s2.reshape(NC, NPP, 128), g2p, ddb_p, b2p, wlbd,
                     blin.reshape(1, 1), be, bo)
    return out.reshape(-1)


# speedup vs baseline: 2.1126x; 1.3293x over previous
import functools

import jax
import jax.numpy as jnp
from jax import lax
from jax.experimental import pallas as pl
from jax.experimental.pallas import tpu as pltpu
from jax.experimental.pallas import tpu_sc as plsc

N = 10000
IN_DIM = 128
D = 64
E = 320000
NG = 64

NC = 2
NS = 16
NW = NC * NS
EPT = E // NW
K = 80
CH = EPT // K
NP = 10240
RPT = NP // NS

BN = 2048
BP = BN // 2
GN = NP // BN
NPP = NP // 2
NPK = N * D // 128

_mesh = plsc.VectorSubcoreMesh(core_axis_name="c", subcore_axis_name="s")


@functools.partial(
    pl.kernel,
    mesh=_mesh,
    out_type=jax.ShapeDtypeStruct((NC, NP, D), jnp.float32),
    scratch_types=[
        pltpu.VMEM((CH, K), jnp.int32),
        pltpu.VMEM((K,), jnp.float32),
        pltpu.VMEM((RPT,), jnp.float32),
        pltpu.VMEM((RPT, D), jnp.float32),
        pltpu.VMEM_SHARED((NP,), jnp.float32),
        pltpu.SemaphoreType.DMA,
    ],
    compiler_params=pltpu.CompilerParams(use_tc_tiling_on_sc=False),
)
def _sc_degree(e_hbm, ones_hbm, zeros_hbm, out_hbm,
               dst_v, ones_v, deg_v, wide_v, acc_sh, hsem):
    cid = lax.axis_index("c")
    sid = lax.axis_index("s")
    wid = cid * NS + sid
    pltpu.sync_copy(e_hbm.at[1, wid], dst_v)
    pltpu.sync_copy(ones_hbm, ones_v)
    pltpu.sync_copy(zeros_hbm.at[pl.ds(sid * RPT, RPT)],
                    acc_sh.at[pl.ds(sid * RPT, RPT)])
    plsc.subcore_barrier()

    def body(j, carry):
        pltpu.async_copy(ones_v, acc_sh.at[dst_v.at[j]], hsem, add=True)
        return carry

    lax.fori_loop(0, CH, body, 0)

    def drain(j, carry):
        pltpu.make_async_copy(ones_v, acc_sh.at[dst_v.at[j]], hsem).wait()
        return carry

    lax.fori_loop(0, CH, drain, 0)
    plsc.subcore_barrier()

    pltpu.sync_copy(acc_sh.at[pl.ds(sid * RPT, RPT)], deg_v)

    def bodyb(t, carry):
        base = t * 16
        v = deg_v[pl.ds(base, 16)]
        for l in range(16):
            row = jnp.full((16,), v[l], jnp.float32)
            for c in range(4):
                wide_v[base + l, pl.ds(16 * c, 16)] = row
        return carry

    lax.fori_loop(0, RPT // 16, bodyb, 0)
    pltpu.sync_copy(wide_v, out_hbm.at[cid, pl.ds(sid * RPT, RPT)])


NBUF = 5


@functools.partial(
    pl.kernel,
    mesh=_mesh,
    out_type=jax.ShapeDtypeStruct((NC, NP, D), jnp.float32),
    scratch_types=[
        pltpu.VMEM((CH, K), jnp.int32),
        pltpu.VMEM((CH, K), jnp.int32),
        pltpu.VMEM((NBUF, K, D), jnp.float32),
        pltpu.VMEM_SHARED((NP, D), jnp.float32),
        [pltpu.SemaphoreType.DMA] * NBUF,
        [pltpu.SemaphoreType.DMA] * NBUF,
    ],
    compiler_params=pltpu.CompilerParams(use_tc_tiling_on_sc=False),
)
def _sc_scatter(g_hbm, e_hbm, zeros_hbm, out_hbm,
                src_v, dst_v, rows_v, acc_sh, gsem, ssem):
    cid = lax.axis_index("c")
    sid = lax.axis_index("s")
    wid = cid * NS + sid
    pltpu.sync_copy(e_hbm.at[0, wid], src_v)
    pltpu.sync_copy(e_hbm.at[1, wid], dst_v)
    for b in range(3):
        pltpu.async_copy(g_hbm.at[src_v.at[b]], rows_v.at[b], gsem[b])
    pltpu.sync_copy(zeros_hbm.at[pl.ds(sid * RPT, RPT)],
                    acc_sh.at[pl.ds(sid * RPT, RPT)])
    plsc.subcore_barrier()

    def body(t, carry):
        for bb in range(NBUF):
            j = NBUF * t + bb
            pltpu.make_async_copy(g_hbm.at[src_v.at[j]], rows_v.at[bb],
                                  gsem[bb]).wait()
            pltpu.async_copy(rows_v.at[bb], acc_sh.at[dst_v.at[j]],
                             ssem[bb], add=True)
            br = (bb + 3) % NBUF

            @pl.when(j + 3 < CH)
            def _(j=j, bb=bb, br=br):
                @pl.when(j >= 2)
                def _():
                    pltpu.make_async_copy(
                        rows_v.at[br], acc_sh.at[dst_v.at[j - 2]],
                        ssem[br]).wait()

                pltpu.async_copy(g_hbm.at[src_v.at[j + 3]], rows_v.at[br],
                                 gsem[br])
        return carry

    lax.fori_loop(0, CH // NBUF, body, 0)
    for jj in range(CH - NBUF, CH):
        bb = jj % NBUF
        pltpu.make_async_copy(rows_v.at[bb], acc_sh.at[dst_v.at[jj]],
                              ssem[bb]).wait()

    plsc.subcore_barrier()
    pltpu.sync_copy(acc_sh.at[pl.ds(sid * RPT, RPT)],
                    out_hbm.at[cid, pl.ds(sid * RPT, RPT)])


def _dinv_packed(deg_ref):
    d = deg_ref[...]
    dd = d[0] + d[1] + 1.0
    return lax.rsqrt(jnp.maximum(dd, 1.0))


def _tc1_body(x_ref, w1_ref, deg_ref, g_ref):
    dinv = _dinv_packed(deg_ref)
    h = jnp.dot(x_ref[...], w1_ref[...], preferred_element_type=jnp.float32)
    g_ref[...] = h * dinv


_tc_stage1 = pl.pallas_call(
    _tc1_body,
    grid=(GN,),
    in_specs=[
        pl.BlockSpec((BP, 2 * IN_DIM), lambda i: (i, 0)),
        pl.BlockSpec((2 * IN_DIM, 128), lambda i: (0, 0)),
        pl.BlockSpec((NC, BP, 128), lambda i: (0, i, 0)),
    ],
    out_specs=pl.BlockSpec((BP, 128), lambda i: (i, 0)),
    out_shape=jax.ShapeDtypeStruct((NPK, 128), jnp.float32),
)


def _tc2_body(s_ref, g1_ref, deg_ref, w2_ref, b1_ref, g2_ref):
    dinv = _dinv_packed(deg_ref)
    sp = s_ref[...]
    s = sp[0] + sp[1]
    t1 = jnp.maximum((s + g1_ref[...]) * dinv + b1_ref[...], 0.0)
    h2 = jnp.dot(t1, w2_ref[...], preferred_element_type=jnp.float32)
    g2_ref[...] = h2 * dinv


_tc_stage2 = pl.pallas_call(
    _tc2_body,
    grid=(GN,),
    in_specs=[
        pl.BlockSpec((NC, BP, 128), lambda i: (0, i, 0)),
        pl.BlockSpec((BP, 128), lambda i: (i, 0)),
        pl.BlockSpec((NC, BP, 128), lambda i: (0, i, 0)),
        pl.BlockSpec((128, 128), lambda i: (0, 0)),
        pl.BlockSpec((1, 128), lambda i: (0, 0)),
    ],
    out_specs=pl.BlockSpec((BP, 128), lambda i: (i, 0)),
    out_shape=jax.ShapeDtypeStruct((NPK, 128), jnp.float32),
)


def _tc3_body(s_ref, g2_ref, deg_ref, b2_ref, wl_ref, bl_ref, be_ref, bo_ref,
              out_ref, acc_ref):
    i = pl.program_id(0)
    dinv = _dinv_packed(deg_ref)
    sp = s_ref[...]
    s = sp[0] + sp[1]
    h = jnp.maximum((s + g2_ref[...]) * dinv + b2_ref[...], 0.0)
    zp = jnp.dot(h, wl_ref[...], preferred_element_type=jnp.float32)
    ones = jnp.ones((BP, 1), jnp.float32)
    giota = lax.broadcasted_iota(jnp.int32, (BP, NG), 1)
    part = jnp.zeros((NG, 2), jnp.float32)
    for b_ref, col in ((be_ref, 0), (bo_ref, 1)):
        b = b_ref[0, 0, :]
        onehot = (b[:, None] == giota).astype(jnp.float32)
        vals = jnp.where(b[:, None] >= 0,
                         jnp.concatenate([zp[:, col:col + 1], ones], axis=1),
                         0.0)
        part = part + lax.dot_general(onehot, vals, (((0,), (0,)), ((), ())),
                                      preferred_element_type=jnp.float32)

    @pl.when(i == 0)
    def _():
        acc_ref[...] = jnp.zeros_like(acc_ref)

    acc_ref[...] += part
    out_ref[0, :] = (acc_ref[:, 0] / jnp.maximum(acc_ref[:, 1], 1.0)
                     + bl_ref[0, 0])


_tc_stage3 = pl.pallas_call(
    _tc3_body,
    grid=(GN,),
    in_specs=[
        pl.BlockSpec((NC, BP, 128), lambda i: (0, i, 0)),
        pl.BlockSpec((BP, 128), lambda i: (i, 0)),
        pl.BlockSpec((NC, BP, 128), lambda i: (0, i, 0)),
        pl.BlockSpec((1, 128), lambda i: (0, 0)),
        pl.BlockSpec((128, 2), lambda i: (0, 0)),
        pl.BlockSpec((1, 1), lambda i: (0, 0)),
        pl.BlockSpec((1, 1, BP), lambda i: (i, 0, 0)),
        pl.BlockSpec((1, 1, BP), lambda i: (i, 0, 0)),
    ],
    out_specs=pl.BlockSpec((1, NG), lambda i: (0, 0)),
    out_shape=jax.ShapeDtypeStruct((1, NG), jnp.float32),
    scratch_shapes=[pltpu.VMEM((NG, 2), jnp.float32)],
)


def _blockdiag2(w):
    r, c = w.shape
    z = jnp.zeros((r, c), w.dtype)
    return jnp.concatenate(
        [jnp.concatenate([w, z], axis=1), jnp.concatenate([z, w], axis=1)],
        axis=0)


def kernel(x, edge_index, batch, W1, b1, W2, b2, Wlin, blin):
    e4 = edge_index.reshape(2, NW, CH, K)
    ones1 = jnp.ones((K,), jnp.float32)
    zer1 = jnp.zeros((NP,), jnp.float32)
    zer64 = jnp.zeros((NP, D), jnp.float32)
    batch_pad = jnp.concatenate([batch, jnp.full((NP - N,), -1, jnp.int32)])
    be = batch_pad[0::2].reshape(GN, 1, BP)
    bo = batch_pad[1::2].reshape(GN, 1, BP)
    w1bd = _blockdiag2(W1)
    w2bd = _blockdiag2(W2)
    wlbd = _blockdiag2(Wlin)
    b1p = jnp.tile(b1, 2).reshape(1, 128)
    b2p = jnp.tile(b2, 2).reshape(1, 128)

    ddb = _sc_degree(e4, ones1, zer1)
    ddb_p = ddb.reshape(NC, NPP, 128)
    g1p = _tc_stage1(x.reshape(N // 2, 2 * IN_DIM), w1bd, ddb_p)
    s1 = _sc_scatter(g1p.reshape(N, D), e4, zer64)
    g2p = _tc_stage2(s1.reshape(NC, NPP, 128), g1p, ddb_p, w2bd, b1p)
    s2 = _sc_scatter(g2p.reshape(N, D), e4, zer64)
    out = _tc_stage3(s2.reshape(NC, NPP, 128), g2p, ddb_p, b2p, wlbd,
                     blin.reshape(1, 1), be, bo)
    return out.reshape(-1)


# speedup vs baseline: 2.2047x; 1.0436x over previous
import functools

import jax
import jax.numpy as jnp
from jax import lax
from jax.experimental import pallas as pl
from jax.experimental.pallas import tpu as pltpu
from jax.experimental.pallas import tpu_sc as plsc

N = 10000
IN_DIM = 128
D = 64
E = 320000
NG = 64

NC = 2
NS = 16
NW = NC * NS
EPT = E // NW
K = 80
CH = EPT // K
NP = 10240
RPT = NP // NS

BN = 2048
BP = BN // 2
GN = NP // BN
NPP = NP // 2
NPK = N * D // 128

_mesh = plsc.VectorSubcoreMesh(core_axis_name="c", subcore_axis_name="s")


@functools.partial(
    pl.kernel,
    mesh=_mesh,
    out_type=jax.ShapeDtypeStruct((NC, NP, D), jnp.float32),
    scratch_types=[
        pltpu.VMEM((CH, K), jnp.int32),
        pltpu.VMEM((K,), jnp.float32),
        pltpu.VMEM((RPT,), jnp.float32),
        pltpu.VMEM((RPT, D), jnp.float32),
        pltpu.VMEM_SHARED((NP,), jnp.float32),
        pltpu.SemaphoreType.DMA,
    ],
    compiler_params=pltpu.CompilerParams(use_tc_tiling_on_sc=False),
)
def _sc_degree(e_hbm, ones_hbm, out_hbm,
               dst_v, ones_v, deg_v, wide_v, acc_sh, hsem):
    cid = lax.axis_index("c")
    sid = lax.axis_index("s")
    wid = cid * NS + sid
    pltpu.sync_copy(e_hbm.at[1, wid], dst_v)
    pltpu.sync_copy(ones_hbm, ones_v)
    zv = jnp.zeros((16,), jnp.float32)

    def zrow(r, carry):
        deg_v[pl.ds(r * 16, 16)] = zv
        return carry

    lax.fori_loop(0, RPT // 16, zrow, 0)
    pltpu.sync_copy(deg_v, acc_sh.at[pl.ds(sid * RPT, RPT)])
    plsc.subcore_barrier()

    def body(j, carry):
        pltpu.async_copy(ones_v, acc_sh.at[dst_v.at[j]], hsem, add=True)
        return carry

    lax.fori_loop(0, CH, body, 0)

    def drain(j, carry):
        pltpu.make_async_copy(ones_v, acc_sh.at[dst_v.at[j]], hsem).wait()
        return carry

    lax.fori_loop(0, CH, drain, 0)
    plsc.subcore_barrier()

    pltpu.sync_copy(acc_sh.at[pl.ds(sid * RPT, RPT)], deg_v)

    def bodyb(t, carry):
        base = t * 16
        v = deg_v[pl.ds(base, 16)]
        for l in range(16):
            row = jnp.full((16,), v[l], jnp.float32)
            for c in range(4):
                wide_v[base + l, pl.ds(16 * c, 16)] = row
        return carry

    lax.fori_loop(0, RPT // 16, bodyb, 0)
    pltpu.sync_copy(wide_v, out_hbm.at[cid, pl.ds(sid * RPT, RPT)])


NBUF = 5


@functools.partial(
    pl.kernel,
    mesh=_mesh,
    out_type=jax.ShapeDtypeStruct((NC, NP, D), jnp.float32),
    scratch_types=[
        pltpu.VMEM((CH, K), jnp.int32),
        pltpu.VMEM((CH, K), jnp.int32),
        pltpu.VMEM((NBUF, K, D), jnp.float32),
        pltpu.VMEM((K, D), jnp.float32),
        pltpu.VMEM_SHARED((NP, D), jnp.float32),
        [pltpu.SemaphoreType.DMA] * NBUF,
        [pltpu.SemaphoreType.DMA] * NBUF,
    ],
    compiler_params=pltpu.CompilerParams(use_tc_tiling_on_sc=False),
)
def _sc_scatter(g_hbm, e_hbm, out_hbm,
                src_v, dst_v, rows_v, zbuf, acc_sh, gsem, ssem):
    cid = lax.axis_index("c")
    sid = lax.axis_index("s")
    wid = cid * NS + sid
    pltpu.sync_copy(e_hbm.at[0, wid], src_v)
    pltpu.sync_copy(e_hbm.at[1, wid], dst_v)
    for b in range(3):
        pltpu.async_copy(g_hbm.at[src_v.at[b]], rows_v.at[b], gsem[b])
    zv = jnp.zeros((16,), jnp.float32)

    def zrow(r, carry):
        for c in range(4):
            zbuf[r, pl.ds(16 * c, 16)] = zv
        return carry

    lax.fori_loop(0, K, zrow, 0)
    for i in range(RPT // K):
        pltpu.sync_copy(zbuf, acc_sh.at[pl.ds(sid * RPT + i * K, K)])
    plsc.subcore_barrier()

    def body(t, carry):
        for bb in range(NBUF):
            j = NBUF * t + bb
            pltpu.make_async_copy(g_hbm.at[src_v.at[j]], rows_v.at[bb],
                                  gsem[bb]).wait()
            pltpu.async_copy(rows_v.at[bb], acc_sh.at[dst_v.at[j]],
                             ssem[bb], add=True)
            br = (bb + 3) % NBUF

            @pl.when(j + 3 < CH)
            def _(j=j, bb=bb, br=br):
                @pl.when(j >= 2)
                def _():
                    pltpu.make_async_copy(
                        rows_v.at[br], acc_sh.at[dst_v.at[j - 2]],
                        ssem[br]).wait()

                pltpu.async_copy(g_hbm.at[src_v.at[j + 3]], rows_v.at[br],
                                 gsem[br])
        return carry

    lax.fori_loop(0, CH // NBUF, body, 0)
    for jj in range(CH - NBUF, CH):
        bb = jj % NBUF
        pltpu.make_async_copy(rows_v.at[bb], acc_sh.at[dst_v.at[jj]],
                              ssem[bb]).wait()

    plsc.subcore_barrier()
    pltpu.sync_copy(acc_sh.at[pl.ds(sid * RPT, RPT)],
                    out_hbm.at[cid, pl.ds(sid * RPT, RPT)])


def _dinv_packed(deg_ref):
    d = deg_ref[...]
    dd = d[0] + d[1] + 1.0
    return lax.rsqrt(jnp.maximum(dd, 1.0))


def _tc1_body(x_ref, w1_ref, deg_ref, g_ref, dinv_ref):
    dinv = _dinv_packed(deg_ref)
    h = jnp.dot(x_ref[...], w1_ref[...], preferred_element_type=jnp.float32)
    g_ref[...] = h * dinv
    dinv_ref[...] = dinv


_tc_stage1 = pl.pallas_call(
    _tc1_body,
    grid=(GN,),
    in_specs=[
        pl.BlockSpec((BP, 2 * IN_DIM), lambda i: (i, 0)),
        pl.BlockSpec((2 * IN_DIM, 128), lambda i: (0, 0)),
        pl.BlockSpec((NC, BP, 128), lambda i: (0, i, 0)),
    ],
    out_specs=[
        pl.BlockSpec((BP, 128), lambda i: (i, 0)),
        pl.BlockSpec((BP, 128), lambda i: (i, 0)),
    ],
    out_shape=[
        jax.ShapeDtypeStruct((NPK, 128), jnp.float32),
        jax.ShapeDtypeStruct((NPK, 128), jnp.float32),
    ],
)


def _tc2_body(s_ref, g1_ref, dinv_ref, w2_ref, b1_ref, g2_ref):
    dinv = dinv_ref[...]
    sp = s_ref[...]
    s = sp[0] + sp[1]
    t1 = jnp.maximum((s + g1_ref[...]) * dinv + b1_ref[...], 0.0)
    h2 = jnp.dot(t1, w2_ref[...], preferred_element_type=jnp.float32)
    g2_ref[...] = h2 * dinv


_tc_stage2 = pl.pallas_call(
    _tc2_body,
    grid=(GN,),
    in_specs=[
        pl.BlockSpec((NC, BP, 128), lambda i: (0, i, 0)),
        pl.BlockSpec((BP, 128), lambda i: (i, 0)),
        pl.BlockSpec((BP, 128), lambda i: (i, 0)),
        pl.BlockSpec((128, 128), lambda i: (0, 0)),
        pl.BlockSpec((1, 128), lambda i: (0, 0)),
    ],
    out_specs=pl.BlockSpec((BP, 128), lambda i: (i, 0)),
    out_shape=jax.ShapeDtypeStruct((NPK, 128), jnp.float32),
)


def _tc3_body(s_ref, g2_ref, dinv_ref, b2_ref, wl_ref, bl_ref, be_ref, bo_ref,
              out_ref, acc_ref):
    i = pl.program_id(0)
    dinv = dinv_ref[...]
    sp = s_ref[...]
    s = sp[0] + sp[1]
    h = jnp.maximum((s + g2_ref[...]) * dinv + b2_ref[...], 0.0)
    zp = jnp.dot(h, wl_ref[...], preferred_element_type=jnp.float32)
    ones = jnp.ones((BP, 1), jnp.float32)
    giota = lax.broadcasted_iota(jnp.int32, (BP, NG), 1)
    part = jnp.zeros((NG, 2), jnp.float32)
    for b_ref, col in ((be_ref, 0), (bo_ref, 1)):
        b = b_ref[0, 0, :]
        onehot = (b[:, None] == giota).astype(jnp.float32)
        vals = jnp.where(b[:, None] >= 0,
                         jnp.concatenate([zp[:, col:col + 1], ones], axis=1),
                         0.0)
        part = part + lax.dot_general(onehot, vals, (((0,), (0,)), ((), ())),
                                      preferred_element_type=jnp.float32)

    @pl.when(i == 0)
    def _():
        acc_ref[...] = jnp.zeros_like(acc_ref)

    acc_ref[...] += part
    out_ref[0, :] = (acc_ref[:, 0] / jnp.maximum(acc_ref[:, 1], 1.0)
                     + bl_ref[0, 0])


_tc_stage3 = pl.pallas_call(
    _tc3_body,
    grid=(GN,),
    in_specs=[
        pl.BlockSpec((NC, BP, 128), lambda i: (0, i, 0)),
        pl.BlockSpec((BP, 128), lambda i: (i, 0)),
        pl.BlockSpec((BP, 128), lambda i: (i, 0)),
        pl.BlockSpec((1, 128), lambda i: (0, 0)),
        pl.BlockSpec((128, 2), lambda i: (0, 0)),
        pl.BlockSpec((1, 1), lambda i: (0, 0)),
        pl.BlockSpec((1, 1, BP), lambda i: (i, 0, 0)),
        pl.BlockSpec((1, 1, BP), lambda i: (i, 0, 0)),
    ],
    out_specs=pl.BlockSpec((1, NG), lambda i: (0, 0)),
    out_shape=jax.ShapeDtypeStruct((1, NG), jnp.float32),
    scratch_shapes=[pltpu.VMEM((NG, 2), jnp.float32)],
)


def _blockdiag2(w):
    r, c = w.shape
    z = jnp.zeros((r, c), w.dtype)
    return jnp.concatenate(
        [jnp.concatenate([w, z], axis=1), jnp.concatenate([z, w], axis=1)],
        axis=0)


def kernel(x, edge_index, batch, W1, b1, W2, b2, Wlin, blin):
    e4 = edge_index.reshape(2, NW, CH, K)
    ones1 = jnp.ones((K,), jnp.float32)
    batch_pad = jnp.concatenate([batch, jnp.full((NP - N,), -1, jnp.int32)])
    be = batch_pad[0::2].reshape(GN, 1, BP)
    bo = batch_pad[1::2].reshape(GN, 1, BP)
    w1bd = _blockdiag2(W1)
    w2bd = _blockdiag2(W2)
    wlbd = _blockdiag2(Wlin)
    b1p = jnp.tile(b1, 2).reshape(1, 128)
    b2p = jnp.tile(b2, 2).reshape(1, 128)

    ddb = _sc_degree(e4, ones1)
    ddb_p = ddb.reshape(NC, NPP, 128)
    g1p, dinvp = _tc_stage1(x.reshape(N // 2, 2 * IN_DIM), w1bd, ddb_p)
    s1 = _sc_scatter(g1p.reshape(N, D), e4)
    g2p = _tc_stage2(s1.reshape(NC, NPP, 128), g1p, dinvp, w2bd, b1p)
    s2 = _sc_scatter(g2p.reshape(N, D), e4)
    out = _tc_stage3(s2.reshape(NC, NPP, 128), g2p, dinvp, b2p, wlbd,
                     blin.reshape(1, 1), be, bo)
    return out.reshape(-1)
